# baseline (speedup 1.0000x reference)
import functools

import jax
import jax.numpy as jnp
from jax import lax
from jax.experimental import pallas as pl
from jax.experimental.pallas import tpu as pltpu
from jax.experimental.pallas import tpu_sc as plsc

_LANES = 128


_UNROLL = 49


def _nn_body(nstep, qb, y_ref, gx_ref, gy_ref, sig_ref, idx_ref, g_ref):
    yx = y_ref[:, 0:1]
    yy = y_ref[:, 1:2]
    lane = lax.broadcasted_iota(jnp.int32, (1, _LANES), 1).astype(jnp.float32)

    def one(c, bv, bi):
        off = pl.multiple_of(c * _LANES, _LANES)
        gx = gx_ref[:, pl.ds(off, _LANES)]
        gy = gy_ref[:, pl.ds(off, _LANES)]
        dx = yx - gx
        dy = yy - gy
        d = dx * dx + dy * dy
        upd = d < bv
        ki = jnp.float32(_LANES) * lax.convert_element_type(c, jnp.float32) + lane
        return jnp.where(upd, d, bv), jnp.where(upd, ki, bi)

    def step(s, carry):
        bv, bi = carry
        for u in range(_UNROLL):
            bv, bi = one(s * _UNROLL + u, bv, bi)
        return bv, bi

    bv0 = jnp.full((qb, _LANES), jnp.inf, jnp.float32)
    bi0 = jnp.zeros((qb, _LANES), jnp.float32)
    bv, bi = lax.fori_loop(0, nstep // _UNROLL, step, (bv0, bi0))

    m = jnp.min(bv, axis=1, keepdims=True)
    big = jnp.float32(3.0e38)
    idxf = jnp.min(jnp.where(bv == m, bi, big), axis=1, keepdims=True)
    idx_ref[...] = idxf.astype(jnp.int32)
    g_ref[...] = jnp.broadcast_to(jnp.exp(sig_ref[0, 0]), g_ref.shape)


def _nn_argmin(y, gxp, gyp, sig):
    q = y.shape[0]
    kpad = gxp.shape[1]
    qb = 256
    nstep = kpad // _LANES
    body = functools.partial(_nn_body, nstep, qb)
    return pl.pallas_call(
        body,
        grid=(q // qb,),
        in_specs=[
            pl.BlockSpec((qb, 2), lambda i: (i, 0)),
            pl.BlockSpec((1, kpad), lambda i: (0, 0)),
            pl.BlockSpec((1, kpad), lambda i: (0, 0)),
            pl.BlockSpec((1, 1), lambda i: (0, 0)),
        ],
        out_specs=[
            pl.BlockSpec((qb, 1), lambda i: (i, 0)),
            pl.BlockSpec((qb, 2), lambda i: (i, 0)),
        ],
        out_shape=[
            jax.ShapeDtypeStruct((q, 1), jnp.int32),
            jax.ShapeDtypeStruct((q, 2), jnp.float32),
        ],
    )(y, gxp, gyp, sig)


def _sc_gather(idx, u_grid, v_grid):
    q = idx.shape[0]
    info = plsc.get_sparse_core_info()
    nw = info.num_cores * info.num_subcores
    bpw = q // nw
    mesh = plsc.VectorSubcoreMesh(core_axis_name="c", subcore_axis_name="s")

    @functools.partial(
        pl.kernel,
        mesh=mesh,
        out_type=[
            jax.ShapeDtypeStruct((q,), jnp.float32),
            jax.ShapeDtypeStruct((q,), jnp.float32),
        ],
        scratch_types=[
            pltpu.VMEM((bpw,), jnp.int32),
            pltpu.VMEM((bpw,), jnp.float32),
            pltpu.VMEM((bpw,), jnp.float32),
            pltpu.SemaphoreType.DMA,
            pltpu.SemaphoreType.DMA,
        ],
    )
    def gather_k(idx_hbm, u_hbm, v_hbm, uo_hbm, vo_hbm, idx_v, u_v, v_v, s1, s2):
        wid = lax.axis_index("s") * info.num_cores + lax.axis_index("c")
        base = wid * bpw
        pltpu.sync_copy(idx_hbm.at[pl.ds(base, bpw)], idx_v)
        cu = pltpu.async_copy(u_hbm.at[idx_v], u_v, s1)
        cv = pltpu.async_copy(v_hbm.at[idx_v], v_v, s2)
        cu.wait()
        cv.wait()
        pltpu.sync_copy(u_v, uo_hbm.at[pl.ds(base, bpw)])
        pltpu.sync_copy(v_v, vo_hbm.at[pl.ds(base, bpw)])

    return gather_k(idx, u_grid, v_grid)


def kernel(y, grid_points, u_grid, v_grid, sigma):
    k = grid_points.shape[0]
    kq = _LANES * _UNROLL
    kpad = (k + kq - 1) // kq * kq
    pad = kpad - k
    gxp = jnp.pad(grid_points[:, 0], (0, pad), constant_values=jnp.inf)
    gyp = jnp.pad(grid_points[:, 1], (0, pad), constant_values=jnp.inf)
    gxp = gxp.reshape(1, kpad)
    gyp = gyp.reshape(1, kpad)
    sig = jnp.reshape(sigma, (1, 1)).astype(jnp.float32)

    idx2, g_out = _nn_argmin(y, gxp, gyp, sig)
    idx = idx2.reshape(y.shape[0])
    u_out, v_out = _sc_gather(idx, u_grid, v_grid)
    f_out = jnp.stack([u_out, v_out], axis=1)
    return f_out, g_out


# speedup vs baseline: 1.2744x; 1.2744x over previous
import functools

import jax
import jax.numpy as jnp
from jax import lax
from jax.experimental import pallas as pl
from jax.experimental.pallas import tpu as pltpu
from jax.experimental.pallas import tpu_sc as plsc

_LANES = 128


_UNROLL = 28


def _nn_body(nstep, qb, y_ref, gx_ref, gy_ref, sig_ref, idx_ref, g_ref):
    yx = y_ref[:, 0:1]
    yy = y_ref[:, 1:2]
    lane = lax.broadcasted_iota(jnp.int32, (1, _LANES), 1).astype(jnp.float32)

    def one(c, bv, bi):
        off = pl.multiple_of(c * _LANES, _LANES)
        gx = gx_ref[:, pl.ds(off, _LANES)]
        gy = gy_ref[:, pl.ds(off, _LANES)]
        dx = yx - gx
        dy = yy - gy
        d = dx * dx + dy * dy
        upd = d < bv
        ki = jnp.float32(_LANES) * lax.convert_element_type(c, jnp.float32) + lane
        return jnp.where(upd, d, bv), jnp.where(upd, ki, bi)

    def step(s, carry):
        bv, bi = carry
        for u in range(_UNROLL):
            bv, bi = one(s * _UNROLL + u, bv, bi)
        return bv, bi

    bv0 = jnp.full((qb, _LANES), jnp.inf, jnp.float32)
    bi0 = jnp.zeros((qb, _LANES), jnp.float32)
    bv, bi = lax.fori_loop(0, nstep // _UNROLL, step, (bv0, bi0))

    m = jnp.min(bv, axis=1, keepdims=True)
    big = jnp.float32(3.0e38)
    idxf = jnp.min(jnp.where(bv == m, bi, big), axis=1, keepdims=True)
    idx_ref[...] = idxf.astype(jnp.int32)
    g_ref[...] = jnp.broadcast_to(jnp.exp(sig_ref[0, 0]), g_ref.shape)


def _nn_argmin(y, gxp, gyp, sig):
    q = y.shape[0]
    kpad = gxp.shape[1]
    qb = 512
    nstep = kpad // _LANES
    body = functools.partial(_nn_body, nstep, qb)
    return pl.pallas_call(
        body,
        grid=(q // qb,),
        in_specs=[
            pl.BlockSpec((qb, 2), lambda i: (i, 0)),
            pl.BlockSpec((1, kpad), lambda i: (0, 0)),
            pl.BlockSpec((1, kpad), lambda i: (0, 0)),
            pl.BlockSpec((1, 1), lambda i: (0, 0)),
        ],
        out_specs=[
            pl.BlockSpec((qb, 1), lambda i: (i, 0)),
            pl.BlockSpec((qb, 2), lambda i: (i, 0)),
        ],
        out_shape=[
            jax.ShapeDtypeStruct((q, 1), jnp.int32),
            jax.ShapeDtypeStruct((q, 2), jnp.float32),
        ],
    )(y, gxp, gyp, sig)


def _sc_gather(idx, u_grid, v_grid):
    q = idx.shape[0]
    info = plsc.get_sparse_core_info()
    nw = info.num_cores * info.num_subcores
    bpw = q // nw
    mesh = plsc.VectorSubcoreMesh(core_axis_name="c", subcore_axis_name="s")

    @functools.partial(
        pl.kernel,
        mesh=mesh,
        out_type=[
            jax.ShapeDtypeStruct((q,), jnp.float32),
            jax.ShapeDtypeStruct((q,), jnp.float32),
        ],
        scratch_types=[
            pltpu.VMEM((bpw,), jnp.int32),
            pltpu.VMEM((bpw,), jnp.float32),
            pltpu.VMEM((bpw,), jnp.float32),
            pltpu.SemaphoreType.DMA,
            pltpu.SemaphoreType.DMA,
        ],
    )
    def gather_k(idx_hbm, u_hbm, v_hbm, uo_hbm, vo_hbm, idx_v, u_v, v_v, s1, s2):
        wid = lax.axis_index("s") * info.num_cores + lax.axis_index("c")
        base = wid * bpw
        pltpu.sync_copy(idx_hbm.at[pl.ds(base, bpw)], idx_v)
        cu = pltpu.async_copy(u_hbm.at[idx_v], u_v, s1)
        cv = pltpu.async_copy(v_hbm.at[idx_v], v_v, s2)
        cu.wait()
        cv.wait()
        pltpu.sync_copy(u_v, uo_hbm.at[pl.ds(base, bpw)])
        pltpu.sync_copy(v_v, vo_hbm.at[pl.ds(base, bpw)])

    return gather_k(idx, u_grid, v_grid)


def kernel(y, grid_points, u_grid, v_grid, sigma):
    k = grid_points.shape[0]
    kq = _LANES * _UNROLL
    kpad = (k + kq - 1) // kq * kq
    pad = kpad - k
    gxp = jnp.pad(grid_points[:, 0], (0, pad), constant_values=jnp.inf)
    gyp = jnp.pad(grid_points[:, 1], (0, pad), constant_values=jnp.inf)
    gxp = gxp.reshape(1, kpad)
    gyp = gyp.reshape(1, kpad)
    sig = jnp.reshape(sigma, (1, 1)).astype(jnp.float32)

    idx2, g_out = _nn_argmin(y, gxp, gyp, sig)
    idx = idx2.reshape(y.shape[0])
    u_out, v_out = _sc_gather(idx, u_grid, v_grid)
    f_out = jnp.stack([u_out, v_out], axis=1)
    return f_out, g_out


# speedup vs baseline: 1.3204x; 1.0361x over previous
import functools

import jax
import jax.numpy as jnp
from jax import lax
from jax.experimental import pallas as pl
from jax.experimental.pallas import tpu as pltpu
from jax.experimental.pallas import tpu_sc as plsc

_LANES = 128


_UNROLL = 28


def _nn_body(nstep, qb, y_ref, gx_ref, gy_ref, sig_ref, idx_ref, g_ref):
    yx = y_ref[:, 0:1]
    yy = y_ref[:, 1:2]
    lane = lax.broadcasted_iota(jnp.int32, (1, _LANES), 1).astype(jnp.float32)

    def one(c, bv, bi):
        off = pl.multiple_of(c * _LANES, _LANES)
        gx = gx_ref[:, pl.ds(off, _LANES)]
        gy = gy_ref[:, pl.ds(off, _LANES)]
        dx = yx - gx
        dy = yy - gy
        d = dx * dx + dy * dy
        upd = d < bv
        ki = jnp.float32(_LANES) * lax.convert_element_type(c, jnp.float32) + lane
        return jnp.where(upd, d, bv), jnp.where(upd, ki, bi)

    def step(s, carry):
        bv, bi = carry
        for u in range(_UNROLL):
            bv, bi = one(s * _UNROLL + u, bv, bi)
        return bv, bi

    bv0 = jnp.full((qb, _LANES), jnp.inf, jnp.float32)
    bi0 = jnp.zeros((qb, _LANES), jnp.float32)
    bv, bi = lax.fori_loop(0, nstep // _UNROLL, step, (bv0, bi0))

    m = jnp.min(bv, axis=1, keepdims=True)
    big = jnp.float32(3.0e38)
    idxf = jnp.min(jnp.where(bv == m, bi, big), axis=1, keepdims=True)
    idx_ref[...] = idxf.astype(jnp.int32)
    g_ref[...] = jnp.broadcast_to(jnp.exp(sig_ref[0, 0]), g_ref.shape)


def _nn_argmin(y, gxp, gyp, sig):
    q = y.shape[0]
    kpad = gxp.shape[1]
    qb = 1024
    nstep = kpad // _LANES
    body = functools.partial(_nn_body, nstep, qb)
    return pl.pallas_call(
        body,
        grid=(q // qb,),
        in_specs=[
            pl.BlockSpec((qb, 2), lambda i: (i, 0)),
            pl.BlockSpec((1, kpad), lambda i: (0, 0)),
            pl.BlockSpec((1, kpad), lambda i: (0, 0)),
            pl.BlockSpec((1, 1), lambda i: (0, 0)),
        ],
        out_specs=[
            pl.BlockSpec((qb, 1), lambda i: (i, 0)),
            pl.BlockSpec((qb, 2), lambda i: (i, 0)),
        ],
        out_shape=[
            jax.ShapeDtypeStruct((q, 1), jnp.int32),
            jax.ShapeDtypeStruct((q, 2), jnp.float32),
        ],
    )(y, gxp, gyp, sig)


def _sc_gather(idx, u_grid, v_grid):
    q = idx.shape[0]
    info = plsc.get_sparse_core_info()
    nw = info.num_cores * info.num_subcores
    bpw = q // nw
    mesh = plsc.VectorSubcoreMesh(core_axis_name="c", subcore_axis_name="s")

    @functools.partial(
        pl.kernel,
        mesh=mesh,
        out_type=[
            jax.ShapeDtypeStruct((q,), jnp.float32),
            jax.ShapeDtypeStruct((q,), jnp.float32),
        ],
        scratch_types=[
            pltpu.VMEM((bpw,), jnp.int32),
            pltpu.VMEM((bpw,), jnp.float32),
            pltpu.VMEM((bpw,), jnp.float32),
            pltpu.SemaphoreType.DMA,
            pltpu.SemaphoreType.DMA,
        ],
    )
    def gather_k(idx_hbm, u_hbm, v_hbm, uo_hbm, vo_hbm, idx_v, u_v, v_v, s1, s2):
        wid = lax.axis_index("s") * info.num_cores + lax.axis_index("c")
        base = wid * bpw
        pltpu.sync_copy(idx_hbm.at[pl.ds(base, bpw)], idx_v)
        cu = pltpu.async_copy(u_hbm.at[idx_v], u_v, s1)
        cv = pltpu.async_copy(v_hbm.at[idx_v], v_v, s2)
        cu.wait()
        cv.wait()
        pltpu.sync_copy(u_v, uo_hbm.at[pl.ds(base, bpw)])
        pltpu.sync_copy(v_v, vo_hbm.at[pl.ds(base, bpw)])

    return gather_k(idx, u_grid, v_grid)


def kernel(y, grid_points, u_grid, v_grid, sigma):
    k = grid_points.shape[0]
    kq = _LANES * _UNROLL
    kpad = (k + kq - 1) // kq * kq
    pad = kpad - k
    gxp = jnp.pad(grid_points[:, 0], (0, pad), constant_values=jnp.inf)
    gyp = jnp.pad(grid_points[:, 1], (0, pad), constant_values=jnp.inf)
    gxp = gxp.reshape(1, kpad)
    gyp = gyp.reshape(1, kpad)
    sig = jnp.reshape(sigma, (1, 1)).astype(jnp.float32)

    idx2, g_out = _nn_argmin(y, gxp, gyp, sig)
    idx = idx2.reshape(y.shape[0])
    u_out, v_out = _sc_gather(idx, u_grid, v_grid)
    f_out = jnp.stack([u_out, v_out], axis=1)
    return f_out, g_out
